# baseline (speedup 1.0000x reference)
import functools

import jax
import jax.numpy as jnp
from jax import lax
from jax.experimental import pallas as pl
from jax.experimental.pallas import tpu as pltpu
from jax.experimental.pallas import tpu_sc as plsc

MAX_DEPTH = 20
_NUM_CORES = 2
_NUM_SUBCORES = 16
_NW = _NUM_CORES * _NUM_SUBCORES


def _table_and_idx_body(spine_smem, spine_row_ref, spine_col_ref, parents_row_ref,
                        emb_ref, pos_ref, table_ref, idx_ref):
    S = spine_row_ref.shape[1]
    D = emb_ref.shape[1]
    emb = emb_ref[...]
    spine_row = spine_row_ref[...]
    parents_row = parents_row_ref[...]
    cur = spine_col_ref[...]
    enc = jnp.zeros((S, D), dtype=jnp.float32)
    plen = jnp.zeros((S, 1), dtype=jnp.int32)
    done = jnp.zeros((S, 1), dtype=jnp.bool_)
    col_iota = lax.broadcasted_iota(jnp.int32, (S, S), 1)
    for _ in range(MAX_DEPTH):
        active = jnp.logical_not(done)
        at_zero = jnp.logical_and(active, cur == 0)
        enc = enc + jnp.where(at_zero, emb[0:1, :], 0.0)
        plen = plen + at_zero.astype(jnp.int32)
        done = jnp.logical_or(done, at_zero)
        step = jnp.logical_and(active, cur != 0)
        cnt = jnp.sum((cur >= spine_row).astype(jnp.int32), axis=1, keepdims=True)
        idx = jnp.clip(cnt - 1, 0, S - 1)
        onehot = (idx == col_iota)
        gathered = jax.lax.dot(onehot.astype(jnp.float32), emb,
                               preferred_element_type=jnp.float32)
        spoint = jnp.sum(jnp.where(onehot, spine_row, 0), axis=1, keepdims=True)
        par = jnp.sum(jnp.where(onehot, parents_row, 0), axis=1, keepdims=True)
        enc = enc + jnp.where(step, gathered, 0.0)
        plen = plen + step.astype(jnp.int32)
        cur = jnp.where(jnp.logical_and(step, spoint > 0), par, cur)
        done = jnp.logical_or(done, jnp.logical_and(step, spoint <= 0))
    norm = jax.lax.rsqrt(jnp.maximum(plen, 1).astype(jnp.float32))
    table_ref[...] = enc * norm

    p = pos_ref[...]
    acc = jnp.zeros(p.shape, dtype=jnp.int32)
    for j in range(S):
        acc = acc + (p >= spine_smem[0, j]).astype(jnp.int32)
    idx_ref[...] = jnp.clip(acc - 1, 0, S - 1)


def _sc_gather_body(table_hbm, idx_hbm, out_hbm, table_v, idx_v, ssem):
    wid = lax.axis_index("s") * _NUM_CORES + lax.axis_index("c")
    nrows = idx_v.shape[0]
    pltpu.sync_copy(table_hbm, table_v)
    pltpu.sync_copy(idx_hbm.at[wid], idx_v)

    def group_body(g, carry):
        ivec = idx_v[pl.ds(g * 16, 16)]
        for j in range(16):
            ridx = ivec[j]
            pltpu.async_copy(table_v.at[ridx], out_hbm.at[wid, g * 16 + j], ssem)
        return carry

    lax.fori_loop(0, nrows // 16, group_body, 0, unroll=False)

    def drain_body(r, carry):
        pltpu.make_async_copy(table_v.at[0], out_hbm.at[wid, 0], ssem).wait()
        return carry

    lax.fori_loop(0, nrows, drain_body, 0, unroll=False)


def kernel(positions, spine, parents, emb):
    B = positions.shape[0]
    S = spine.shape[0]
    D = emb.shape[1]
    b_per_w = B // _NW
    CH = 32
    NCH = b_per_w // CH

    spine32 = spine.astype(jnp.int32)
    parents32 = parents.astype(jnp.int32)
    pos2d = positions.astype(jnp.int32).reshape(B // 2048, 2048)

    table, idx2d = pl.pallas_call(
        _table_and_idx_body,
        out_shape=(
            jax.ShapeDtypeStruct((S, D), jnp.float32),
            jax.ShapeDtypeStruct(pos2d.shape, jnp.int32),
        ),
        in_specs=[
            pl.BlockSpec(memory_space=pltpu.SMEM),
            pl.BlockSpec(memory_space=pltpu.VMEM),
            pl.BlockSpec(memory_space=pltpu.VMEM),
            pl.BlockSpec(memory_space=pltpu.VMEM),
            pl.BlockSpec(memory_space=pltpu.VMEM),
            pl.BlockSpec(memory_space=pltpu.VMEM),
        ],
        out_specs=(
            pl.BlockSpec(memory_space=pltpu.VMEM),
            pl.BlockSpec(memory_space=pltpu.VMEM),
        ),
    )(
        spine32.reshape(1, S),
        spine32.reshape(1, S),
        spine32.reshape(S, 1),
        parents32.reshape(1, S),
        emb,
        pos2d,
    )

    idx_w = idx2d.reshape(_NW, b_per_w)

    mesh = plsc.VectorSubcoreMesh(
        core_axis_name="c", subcore_axis_name="s",
        num_cores=_NUM_CORES, num_subcores=_NUM_SUBCORES)
    out3d = pl.kernel(
        _sc_gather_body,
        out_type=jax.ShapeDtypeStruct((_NW, b_per_w, D), jnp.float32),
        mesh=mesh,
        scratch_types=[
            pltpu.VMEM((S, D), jnp.float32),
            pltpu.VMEM((b_per_w,), jnp.int32),
            pltpu.SemaphoreType.DMA,
        ],
    )(table, idx_w)

    return out3d.reshape(B, D)


# speedup vs baseline: 1.0628x; 1.0628x over previous
import functools

import jax
import jax.numpy as jnp
from jax import lax
from jax.experimental import pallas as pl
from jax.experimental.pallas import tpu as pltpu
from jax.experimental.pallas import tpu_sc as plsc

MAX_DEPTH = 20
_NUM_CORES = 2
_NUM_SUBCORES = 16
_NW = _NUM_CORES * _NUM_SUBCORES


def _table_and_idx_body(spine_smem, spine_row_ref, spine_col_ref, parents_row_ref,
                        emb_ref, pos_ref, table_ref, idx_ref):
    S = spine_row_ref.shape[1]
    D = emb_ref.shape[1]
    emb = emb_ref[...]
    spine_row = spine_row_ref[...]
    parents_row = parents_row_ref[...]
    cur = spine_col_ref[...]
    enc = jnp.zeros((S, D), dtype=jnp.float32)
    plen = jnp.zeros((S, 1), dtype=jnp.int32)
    done = jnp.zeros((S, 1), dtype=jnp.bool_)
    col_iota = lax.broadcasted_iota(jnp.int32, (S, S), 1)
    for _ in range(MAX_DEPTH):
        active = jnp.logical_not(done)
        at_zero = jnp.logical_and(active, cur == 0)
        enc = enc + jnp.where(at_zero, emb[0:1, :], 0.0)
        plen = plen + at_zero.astype(jnp.int32)
        done = jnp.logical_or(done, at_zero)
        step = jnp.logical_and(active, cur != 0)
        cnt = jnp.sum((cur >= spine_row).astype(jnp.int32), axis=1, keepdims=True)
        idx = jnp.clip(cnt - 1, 0, S - 1)
        onehot = (idx == col_iota)
        gathered = jax.lax.dot(onehot.astype(jnp.float32), emb,
                               preferred_element_type=jnp.float32)
        spoint = jnp.sum(jnp.where(onehot, spine_row, 0), axis=1, keepdims=True)
        par = jnp.sum(jnp.where(onehot, parents_row, 0), axis=1, keepdims=True)
        enc = enc + jnp.where(step, gathered, 0.0)
        plen = plen + step.astype(jnp.int32)
        cur = jnp.where(jnp.logical_and(step, spoint > 0), par, cur)
        done = jnp.logical_or(done, jnp.logical_and(step, spoint <= 0))
    norm = jax.lax.rsqrt(jnp.maximum(plen, 1).astype(jnp.float32))
    table_ref[...] = enc * norm

    p = pos_ref[...]
    acc = jnp.zeros(p.shape, dtype=jnp.int32)
    for j in range(S):
        acc = acc + (p >= spine_smem[0, j]).astype(jnp.int32)
    idx_ref[...] = jnp.clip(acc - 1, 0, S - 1)


def _sc_gather_body(table_hbm, idx_hbm, out_hbm, table_v, idx_v, ssem):
    wid = lax.axis_index("s") * _NUM_CORES + lax.axis_index("c")
    nrows = idx_v.shape[0]
    pltpu.sync_copy(table_hbm, table_v)
    pltpu.sync_copy(idx_hbm.at[wid], idx_v)

    def group_body(g, carry):
        ivec = idx_v[pl.ds(g * 16, 16)]
        for j in range(16):
            ridx = ivec[j]
            pltpu.async_copy(table_v.at[ridx], out_hbm.at[wid, g * 16 + j], ssem)
        return carry

    lax.fori_loop(0, nrows // 16, group_body, 0, unroll=False)

    def drain_body(r, carry):
        pltpu.make_async_copy(table_v, out_hbm.at[wid, pl.ds(0, 16)], ssem).wait()
        return carry

    lax.fori_loop(0, nrows // 16, drain_body, 0, unroll=False)


def kernel(positions, spine, parents, emb):
    B = positions.shape[0]
    S = spine.shape[0]
    D = emb.shape[1]
    b_per_w = B // _NW
    CH = 32
    NCH = b_per_w // CH

    spine32 = spine.astype(jnp.int32)
    parents32 = parents.astype(jnp.int32)
    pos2d = positions.astype(jnp.int32).reshape(B // 2048, 2048)

    table, idx2d = pl.pallas_call(
        _table_and_idx_body,
        out_shape=(
            jax.ShapeDtypeStruct((S, D), jnp.float32),
            jax.ShapeDtypeStruct(pos2d.shape, jnp.int32),
        ),
        in_specs=[
            pl.BlockSpec(memory_space=pltpu.SMEM),
            pl.BlockSpec(memory_space=pltpu.VMEM),
            pl.BlockSpec(memory_space=pltpu.VMEM),
            pl.BlockSpec(memory_space=pltpu.VMEM),
            pl.BlockSpec(memory_space=pltpu.VMEM),
            pl.BlockSpec(memory_space=pltpu.VMEM),
        ],
        out_specs=(
            pl.BlockSpec(memory_space=pltpu.VMEM),
            pl.BlockSpec(memory_space=pltpu.VMEM),
        ),
    )(
        spine32.reshape(1, S),
        spine32.reshape(1, S),
        spine32.reshape(S, 1),
        parents32.reshape(1, S),
        emb,
        pos2d,
    )

    idx_w = idx2d.reshape(_NW, b_per_w)

    mesh = plsc.VectorSubcoreMesh(
        core_axis_name="c", subcore_axis_name="s",
        num_cores=_NUM_CORES, num_subcores=_NUM_SUBCORES)
    out3d = pl.kernel(
        _sc_gather_body,
        out_type=jax.ShapeDtypeStruct((_NW, b_per_w, D), jnp.float32),
        mesh=mesh,
        scratch_types=[
            pltpu.VMEM((S, D), jnp.float32),
            pltpu.VMEM((b_per_w,), jnp.int32),
            pltpu.SemaphoreType.DMA,
        ],
    )(table, idx_w)

    return out3d.reshape(B, D)


# speedup vs baseline: 1.0651x; 1.0021x over previous
import jax
import jax.numpy as jnp
from jax import lax
from jax.experimental import pallas as pl
from jax.experimental.pallas import tpu as pltpu
from jax.experimental.pallas import tpu_sc as plsc

MAX_DEPTH = 20
_NUM_CORES = 2
_NUM_SUBCORES = 16
_NW = _NUM_CORES * _NUM_SUBCORES


def _table_and_idx_body(spine_smem, spine_row_ref, spine_col_ref, parents_row_ref,
                        emb_ref, pos_ref, table_ref, idx_ref):
    S = spine_row_ref.shape[1]
    D = emb_ref.shape[1]
    emb = emb_ref[...]
    spine_row = spine_row_ref[...]
    parents_row = parents_row_ref[...]
    cur = spine_col_ref[...]
    enc = jnp.zeros((S, D), dtype=jnp.float32)
    plen = jnp.zeros((S, 1), dtype=jnp.int32)
    done = jnp.zeros((S, 1), dtype=jnp.bool_)
    col_iota = lax.broadcasted_iota(jnp.int32, (S, S), 1)
    for _ in range(MAX_DEPTH):
        active = jnp.logical_not(done)
        at_zero = jnp.logical_and(active, cur == 0)
        enc = enc + jnp.where(at_zero, emb[0:1, :], 0.0)
        plen = plen + at_zero.astype(jnp.int32)
        done = jnp.logical_or(done, at_zero)
        step = jnp.logical_and(active, cur != 0)
        cnt = jnp.sum((cur >= spine_row).astype(jnp.int32), axis=1, keepdims=True)
        idx = jnp.clip(cnt - 1, 0, S - 1)
        onehot = (idx == col_iota)
        gathered = jax.lax.dot(onehot.astype(jnp.float32), emb,
                               preferred_element_type=jnp.float32)
        spoint = jnp.sum(jnp.where(onehot, spine_row, 0), axis=1, keepdims=True)
        par = jnp.sum(jnp.where(onehot, parents_row, 0), axis=1, keepdims=True)
        enc = enc + jnp.where(step, gathered, 0.0)
        plen = plen + step.astype(jnp.int32)
        cur = jnp.where(jnp.logical_and(step, spoint > 0), par, cur)
        done = jnp.logical_or(done, jnp.logical_and(step, spoint <= 0))
    norm = jax.lax.rsqrt(jnp.maximum(plen, 1).astype(jnp.float32))
    table_ref[...] = enc * norm

    p = pos_ref[...]
    acc = jnp.zeros(p.shape, dtype=jnp.int32)
    for j in range(S):
        acc = acc + (p >= spine_smem[0, j]).astype(jnp.int32)
    idx_ref[...] = jnp.clip(acc - 1, 0, S - 1)


def _sc_gather_body(table_hbm, idx_hbm, out_hbm, table_v, idx_v, ssem):
    wid = lax.axis_index("s") * _NUM_CORES + lax.axis_index("c")
    nrows = idx_v.shape[0]
    pltpu.sync_copy(table_hbm, table_v)
    pltpu.sync_copy(idx_hbm.at[wid], idx_v)

    def group_body(g, carry):
        ivec = idx_v[pl.ds(g * 16, 16)]
        for j in range(16):
            ridx = ivec[j]
            pltpu.async_copy(table_v.at[ridx], out_hbm.at[wid, g * 16 + j], ssem)
        return carry

    lax.fori_loop(0, nrows // 16, group_body, 0, unroll=False)

    nspb = table_v.shape[0]

    def drain_body(r, carry):
        pltpu.make_async_copy(table_v, out_hbm.at[wid, pl.ds(0, nspb)], ssem).wait()
        return carry

    lax.fori_loop(0, nrows // nspb, drain_body, 0, unroll=False)


def kernel(positions, spine, parents, emb):
    B = positions.shape[0]
    S = spine.shape[0]
    D = emb.shape[1]
    b_per_w = B // _NW

    spine32 = spine.astype(jnp.int32)
    parents32 = parents.astype(jnp.int32)
    pos2d = positions.astype(jnp.int32).reshape(B // 2048, 2048)

    table, idx2d = pl.pallas_call(
        _table_and_idx_body,
        out_shape=(
            jax.ShapeDtypeStruct((S, D), jnp.float32),
            jax.ShapeDtypeStruct(pos2d.shape, jnp.int32),
        ),
        in_specs=[
            pl.BlockSpec(memory_space=pltpu.SMEM),
            pl.BlockSpec(memory_space=pltpu.VMEM),
            pl.BlockSpec(memory_space=pltpu.VMEM),
            pl.BlockSpec(memory_space=pltpu.VMEM),
            pl.BlockSpec(memory_space=pltpu.VMEM),
            pl.BlockSpec(memory_space=pltpu.VMEM),
        ],
        out_specs=(
            pl.BlockSpec(memory_space=pltpu.VMEM),
            pl.BlockSpec(memory_space=pltpu.VMEM),
        ),
    )(
        spine32.reshape(1, S),
        spine32.reshape(1, S),
        spine32.reshape(S, 1),
        parents32.reshape(1, S),
        emb,
        pos2d,
    )

    idx_w = idx2d.reshape(_NW, b_per_w)

    mesh = plsc.VectorSubcoreMesh(
        core_axis_name="c", subcore_axis_name="s",
        num_cores=_NUM_CORES, num_subcores=_NUM_SUBCORES)
    out3d = pl.kernel(
        _sc_gather_body,
        out_type=jax.ShapeDtypeStruct((_NW, b_per_w, D), jnp.float32),
        mesh=mesh,
        scratch_types=[
            pltpu.VMEM((S, D), jnp.float32),
            pltpu.VMEM((b_per_w,), jnp.int32),
            pltpu.SemaphoreType.DMA,
        ],
    )(table, idx_w)

    return out3d.reshape(B, D)
